# baseline (speedup 1.0000x reference)
import functools

import jax
import jax.numpy as jnp
from jax import lax
from jax.experimental import pallas as pl
from jax.experimental.pallas import tpu as pltpu
from jax.experimental.pallas import tpu_sc as plsc

B = 16
N = 32
K = 4
H = 256
W = 256
HB = H // 2
SR = 16
Q = HB // SR
NC = 2
NS = 16


def _sc_kernel(scores, table):
    mesh = plsc.VectorSubcoreMesh(core_axis_name="c", subcore_axis_name="s")

    @functools.partial(
        pl.kernel,
        mesh=mesh,
        compiler_params=pltpu.CompilerParams(needs_layout_passes=False),
        out_type=jax.ShapeDtypeStruct((B * H, W), jnp.float32),
        scratch_types=[
            pltpu.VMEM((N,), jnp.float32),
            pltpu.VMEM((Q * K, SR), jnp.int32),
            pltpu.VMEM((3, K, SR, W), jnp.float32),
            pltpu.VMEM((2, SR, W), jnp.float32),
            pltpu.SemaphoreType.DMA,
            pltpu.SemaphoreType.DMA,
            pltpu.SemaphoreType.DMA,
            pltpu.SemaphoreType.DMA,
            pltpu.SemaphoreType.DMA,
        ],
    )
    def kern(sc_hbm, table_hbm, out_hbm, sv, idx_v, stg, obuf, g0, g1, g2, w0, w1):
        wid = lax.axis_index("s") * NC + lax.axis_index("c")
        b = wid // 2
        h = wid % 2
        gsem = (g0, g1, g2)
        wsem = (w0, w1)

        pltpu.sync_copy(sc_hbm.at[pl.ds(b * N, N)], sv)
        s0 = sv[pl.ds(0, 16)]
        s1 = sv[pl.ds(16, 16)]
        lanes = lax.iota(jnp.int32, 16)
        neg = jnp.full((16,), -jnp.inf, jnp.float32)

        def maxsplat(x):
            return plsc.cummax(lax.rev(plsc.cummax(x), (0,)))

        picks = []
        for _ in range(K):
            m = jnp.maximum(maxsplat(s0), maxsplat(s1))
            eq0 = s0 == m
            eq1 = s1 == m
            in0 = plsc.all_reduce_population_count(eq0) > 0
            f0 = plsc.all_reduce_ffs(eq0)
            f1 = plsc.all_reduce_ffs(eq1) + 16
            n = jnp.where(in0, f0, f1)
            picks.append(n)
            s0 = jnp.where(jnp.logical_and(in0, lanes == n), neg, s0)
            s1 = jnp.where(lanes == (n - 16), neg, s1)

        base = (b * N) * H + h * HB
        for q in range(Q):
            for k in range(K):
                v = base + picks[k] * H + (q * SR + lanes)
                for pp in range(SR // 16):
                    idx_v[q * K + k, pl.ds(pp * 16, 16)] = v + pp * 16

        def gather(q, s):
            return [
                pltpu.async_copy(
                    table_hbm.at[idx_v.at[q * K + k]], stg.at[s, k], gsem[s])
                for k in range(K)
            ]

        gd = {0: gather(0, 0), 1: gather(1, 1), 2: gather(2, 2)}
        wb = {}
        for q in range(Q):
            s = q % 3
            so = q % 2
            for c in gd.pop(q):
                c.wait()
            if q - 2 in wb:
                wb.pop(q - 2).wait()

            def body(i, _):
                for cc in range(W // 16):
                    sl = pl.ds(cc * 16, 16)
                    obuf[so, i, sl] = (
                        (stg[s, 0, i, sl] + stg[s, 1, i, sl])
                        + (stg[s, 2, i, sl] + stg[s, 3, i, sl])) * 0.25
                return 0

            lax.fori_loop(0, SR, body, 0)
            if q + 3 < Q:
                gd[q + 3] = gather(q + 3, s)
            dst = out_hbm.at[pl.ds(b * H + h * HB + q * SR, SR)]
            wb[q] = pltpu.async_copy(obuf.at[so], dst, wsem[so])
        for q in (Q - 2, Q - 1):
            wb.pop(q).wait()

    return kern(scores, table)


def kernel(iou_scores, mask_preds):
    table = mask_preds.reshape(B * N * H, W)
    out = _sc_kernel(iou_scores.reshape(B * N), table)
    return out.reshape(B, 1, H, W)


# speedup vs baseline: 1.1750x; 1.1750x over previous
import functools

import jax
import jax.numpy as jnp
from jax import lax
from jax.experimental import pallas as pl
from jax.experimental.pallas import tpu as pltpu
from jax.experimental.pallas import tpu_sc as plsc

B = 16
N = 32
K = 4
H = 256
W = 256
HB = H // 2
SR = 32
Q = HB // SR
NC = 2
NS = 16


def _sc_kernel(scores, table):
    mesh = plsc.VectorSubcoreMesh(core_axis_name="c", subcore_axis_name="s")

    @functools.partial(
        pl.kernel,
        mesh=mesh,
        compiler_params=pltpu.CompilerParams(needs_layout_passes=False),
        out_type=jax.ShapeDtypeStruct((B * H, W), jnp.float32),
        scratch_types=[
            pltpu.VMEM((N,), jnp.float32),
            pltpu.VMEM((Q * K, SR), jnp.int32),
            pltpu.VMEM((3, K, SR, W), jnp.float32),
            pltpu.VMEM((2, SR, W), jnp.float32),
            pltpu.SemaphoreType.DMA,
            pltpu.SemaphoreType.DMA,
            pltpu.SemaphoreType.DMA,
            pltpu.SemaphoreType.DMA,
            pltpu.SemaphoreType.DMA,
        ],
    )
    def kern(sc_hbm, table_hbm, out_hbm, sv, idx_v, stg, obuf, g0, g1, g2, w0, w1):
        wid = lax.axis_index("s") * NC + lax.axis_index("c")
        b = wid // 2
        h = wid % 2
        gsem = (g0, g1, g2)
        wsem = (w0, w1)

        pltpu.sync_copy(sc_hbm.at[pl.ds(b * N, N)], sv)
        s0 = sv[pl.ds(0, 16)]
        s1 = sv[pl.ds(16, 16)]
        lanes = lax.iota(jnp.int32, 16)
        neg = jnp.full((16,), -jnp.inf, jnp.float32)

        def maxsplat(x):
            return plsc.cummax(lax.rev(plsc.cummax(x), (0,)))

        picks = []
        for _ in range(K):
            m = jnp.maximum(maxsplat(s0), maxsplat(s1))
            eq0 = s0 == m
            eq1 = s1 == m
            in0 = plsc.all_reduce_population_count(eq0) > 0
            f0 = plsc.all_reduce_ffs(eq0)
            f1 = plsc.all_reduce_ffs(eq1) + 16
            n = jnp.where(in0, f0, f1)
            picks.append(n)
            s0 = jnp.where(jnp.logical_and(in0, lanes == n), neg, s0)
            s1 = jnp.where(lanes == (n - 16), neg, s1)

        base = (b * N) * H + h * HB
        for q in range(Q):
            for k in range(K):
                v = base + picks[k] * H + (q * SR + lanes)
                idx_v[q * K + k, pl.ds(0, 16)] = v
                idx_v[q * K + k, pl.ds(16, 16)] = v + 16

        def gather(q, s):
            return [
                pltpu.async_copy(
                    table_hbm.at[idx_v.at[q * K + k]], stg.at[s, k], gsem[s])
                for k in range(K)
            ]

        gd = {0: gather(0, 0), 1: gather(1, 1), 2: gather(2, 2)}
        wb = {}
        for q in range(Q):
            s = q % 3
            so = q % 2
            for c in gd.pop(q):
                c.wait()
            if q - 2 in wb:
                wb.pop(q - 2).wait()

            def body(i, _):
                for cc in range(W // 16):
                    sl = pl.ds(cc * 16, 16)
                    obuf[so, i, sl] = (
                        (stg[s, 0, i, sl] + stg[s, 1, i, sl])
                        + (stg[s, 2, i, sl] + stg[s, 3, i, sl])) * 0.25
                return 0

            lax.fori_loop(0, SR, body, 0)
            if q + 3 < Q:
                gd[q + 3] = gather(q + 3, s)
            dst = out_hbm.at[pl.ds(b * H + h * HB + q * SR, SR)]
            wb[q] = pltpu.async_copy(obuf.at[so], dst, wsem[so])
        for q in (Q - 2, Q - 1):
            wb.pop(q).wait()

    return kern(scores, table)


def kernel(iou_scores, mask_preds):
    table = mask_preds.reshape(B * N * H, W)
    out = _sc_kernel(iou_scores.reshape(B * N), table)
    return out.reshape(B, 1, H, W)
